# baseline (speedup 1.0000x reference)
import math

import jax
import jax.numpy as jnp
from jax import lax
from jax.experimental import pallas as pl
from jax.experimental.pallas import tpu as pltpu
from jax.experimental.pallas import tpu_sc as plsc

D_MODEL = 64
SCALE = math.sqrt(D_MODEL)
NC, NS = 2, 16
NW = NC * NS
LANES = 16
CHUNK = 128
NBUF = 4


def _embed_body(x_hbm, lut_hbm, out_hbm, idx_v, grows, wrows, gsem, wsem):
    wid = lax.axis_index("s") * NC + lax.axis_index("c")
    n_chunks = x_hbm.shape[0] // NW
    q0 = wid * n_chunks

    pltpu.sync_copy(x_hbm.at[pl.ds(q0, n_chunks), :], idx_v)

    for b in range(NBUF):
        pltpu.async_copy(lut_hbm.at[idx_v.at[b]], grows.at[b], gsem.at[b])

    iotas = [lax.iota(jnp.int32, LANES) + cc0 for cc0 in range(0, CHUNK, LANES)]

    def outer(t, carry):
        for b in range(NBUF):
            ql = NBUF * t + b
            q = q0 + ql
            d1 = lax.shift_right_logical(q, 5)
            tc = lax.bitwise_and(q, 31)

            pltpu.make_async_copy(
                lut_hbm.at[idx_v.at[b]], grows.at[b], gsem.at[b]).wait()

            @pl.when(ql >= NBUF)
            def _():
                pltpu.make_async_copy(
                    wrows.at[b], out_hbm.at[pl.ds(0, 8), 0, :, :],
                    wsem.at[b]).wait()

            def scale_k(k, c):
                for rr in range(8):
                    col = jnp.full((LANES,), 8 * k + rr, jnp.int32)
                    for ci, rows in enumerate(iotas):
                        v = plsc.load_gather(grows.at[b], [rows, col])
                        wrows[b, k, rr, pl.ds(ci * LANES, LANES)] = v * SCALE
                return c

            lax.fori_loop(0, 8, scale_k, 0)

            pltpu.async_copy(
                wrows.at[b], out_hbm.at[pl.ds(d1 * 8, 8), tc, :, :],
                wsem.at[b])

            @pl.when(ql + NBUF < n_chunks)
            def _():
                pltpu.async_copy(
                    lut_hbm.at[idx_v.at[ql + NBUF]], grows.at[b], gsem.at[b])
        return carry

    lax.fori_loop(0, n_chunks // NBUF, outer, 0)

    for b in range(NBUF):
        pltpu.make_async_copy(
            wrows.at[b], out_hbm.at[pl.ds(0, 8), 0, :, :], wsem.at[b]).wait()


def kernel(x, lut):
    S, T = x.shape
    xq = x.T.reshape(T * S // CHUNK, CHUNK)
    k = pl.kernel(
        _embed_body,
        out_type=jax.ShapeDtypeStruct((T * 8, S // CHUNK, 8, CHUNK),
                                      jnp.float32),
        mesh=plsc.VectorSubcoreMesh(core_axis_name="c", subcore_axis_name="s"),
        scratch_types=[
            pltpu.VMEM((T * S // (CHUNK * NW), CHUNK), jnp.int32),
            pltpu.VMEM((NBUF, CHUNK, D_MODEL), jnp.float32),
            pltpu.VMEM((NBUF, 8, 8, CHUNK), jnp.float32),
            pltpu.SemaphoreType.DMA((NBUF,)),
            pltpu.SemaphoreType.DMA((NBUF,)),
        ],
        compiler_params=pltpu.CompilerParams(use_tc_tiling_on_sc=False,
                                             needs_layout_passes=False),
    )
    out4 = k(xq, lut)
    return (out4.reshape(T, 8, S // CHUNK, 8, CHUNK)
            .transpose(2, 4, 0, 1, 3)
            .reshape(S, T, D_MODEL))


# speedup vs baseline: 2.6097x; 2.6097x over previous
import math

import jax
import jax.numpy as jnp
from jax import lax
from jax.experimental import pallas as pl
from jax.experimental.pallas import tpu as pltpu
from jax.experimental.pallas import tpu_sc as plsc

D_MODEL = 64
SCALE = math.sqrt(D_MODEL)
NC, NS = 2, 16
NW = NC * NS
LANES = 16
CHUNK = 128
NBUF = 4


def _embed_body(x_hbm, lut_hbm, out_hbm, idx_v, grows, wrows, gsem, wsem):
    wid = lax.axis_index("s") * NC + lax.axis_index("c")
    n_chunks = x_hbm.shape[0] // NW
    q0 = wid * n_chunks

    pltpu.sync_copy(x_hbm.at[pl.ds(q0, n_chunks), :], idx_v)

    for b in range(NBUF):
        pltpu.async_copy(lut_hbm.at[idx_v.at[b]], grows.at[b], gsem.at[b])

    iota = lax.iota(jnp.int32, LANES)
    kvecs = [lax.shift_right_logical(iota + 16 * j, 3) for j in range(4)]
    rvecs = [lax.bitwise_and(iota + 16 * j, 7) for j in range(4)]
    bvecs = [jnp.full((LANES,), b, jnp.int32) for b in range(NBUF)]

    def outer(t, carry):
        for b in range(NBUF):
            ql = NBUF * t + b
            q = q0 + ql
            d1 = lax.shift_right_logical(q, 5)
            tc = lax.bitwise_and(q, 31)

            pltpu.make_async_copy(
                lut_hbm.at[idx_v.at[b]], grows.at[b], gsem.at[b]).wait()

            @pl.when(ql >= NBUF)
            def _():
                pltpu.make_async_copy(
                    wrows.at[b, :, :, pl.ds(0, CHUNK)],
                    out_hbm.at[pl.ds(0, 8), 0, :, :],
                    wsem.at[b]).wait()

            @plsc.parallel_loop(0, CHUNK, step=1, unroll=2)
            def _(i):
                cc = jnp.full((LANES,), i, jnp.int32)
                for j in range(4):
                    v = grows[b, i, pl.ds(16 * j, LANES)] * SCALE
                    plsc.store_scatter(
                        wrows, [bvecs[b], kvecs[j], rvecs[j], cc], v)

            pltpu.async_copy(
                wrows.at[b, :, :, pl.ds(0, CHUNK)],
                out_hbm.at[pl.ds(d1 * 8, 8), tc, :, :],
                wsem.at[b])

            @pl.when(ql + NBUF < n_chunks)
            def _():
                pltpu.async_copy(
                    lut_hbm.at[idx_v.at[ql + NBUF]], grows.at[b], gsem.at[b])
        return carry

    lax.fori_loop(0, n_chunks // NBUF, outer, 0)

    for b in range(NBUF):
        pltpu.make_async_copy(
            wrows.at[b, :, :, pl.ds(0, CHUNK)],
            out_hbm.at[pl.ds(0, 8), 0, :, :], wsem.at[b]).wait()


def kernel(x, lut):
    S, T = x.shape
    xq = x.T.reshape(T * S // CHUNK, CHUNK)
    k = pl.kernel(
        _embed_body,
        out_type=jax.ShapeDtypeStruct((T * 8, S // CHUNK, 8, CHUNK),
                                      jnp.float32),
        mesh=plsc.VectorSubcoreMesh(core_axis_name="c", subcore_axis_name="s"),
        scratch_types=[
            pltpu.VMEM((T * S // (CHUNK * NW), CHUNK), jnp.int32),
            pltpu.VMEM((NBUF, CHUNK, D_MODEL), jnp.float32),
            pltpu.VMEM((NBUF, 8, 8, CHUNK + 1), jnp.float32),
            pltpu.SemaphoreType.DMA((NBUF,)),
            pltpu.SemaphoreType.DMA((NBUF,)),
        ],
        compiler_params=pltpu.CompilerParams(use_tc_tiling_on_sc=False,
                                             needs_layout_passes=False),
    )
    out4 = k(xq, lut)
    return (out4.reshape(T, 8, S // CHUNK, 8, CHUNK)
            .transpose(2, 4, 0, 1, 3)
            .reshape(S, T, D_MODEL))


# speedup vs baseline: 3.2292x; 1.2374x over previous
import math

import jax
import jax.numpy as jnp
from jax import lax
from jax.experimental import pallas as pl
from jax.experimental.pallas import tpu as pltpu
from jax.experimental.pallas import tpu_sc as plsc

D_MODEL = 64
SCALE = math.sqrt(D_MODEL)
NC, NS = 2, 16
NW = NC * NS
LANES = 16
CHUNK = 128
NBUF = 4


def _embed_body(x_hbm, lut_hbm, out_hbm, idx_v, grows, wrows, gsem, wsem):
    wid = lax.axis_index("s") * NC + lax.axis_index("c")
    n_chunks = x_hbm.shape[0] // NW
    q0 = wid * n_chunks

    pltpu.sync_copy(x_hbm.at[pl.ds(q0, n_chunks), :], idx_v)

    for b in range(NBUF):
        pltpu.async_copy(lut_hbm.at[idx_v.at[b]], grows.at[b], gsem.at[b])

    iota = lax.iota(jnp.int32, LANES)
    kvecs = [lax.shift_right_logical(iota + 16 * j, 3) for j in range(4)]
    rvecs = [lax.bitwise_and(iota + 16 * j, 7) for j in range(4)]
    bvecs = [jnp.full((LANES,), b, jnp.int32) for b in range(NBUF)]

    def outer(t, carry):
        for b in range(NBUF):
            ql = NBUF * t + b
            q = q0 + ql
            d1 = lax.shift_right_logical(q, 5)
            tc = lax.bitwise_and(q, 31)

            pltpu.make_async_copy(
                lut_hbm.at[idx_v.at[b]], grows.at[b], gsem.at[b]).wait()

            @pl.when(ql >= NBUF)
            def _():
                pltpu.make_async_copy(
                    wrows.at[b, :, :, pl.ds(0, CHUNK)],
                    out_hbm.at[pl.ds(0, 8), 0, :, :],
                    wsem.at[b]).wait()

            @plsc.parallel_loop(0, CHUNK, step=1, unroll=2)
            def _(i):
                cc = jnp.full((LANES,), i, jnp.int32)
                for j in range(4):
                    v = grows[b, i, pl.ds(16 * j, LANES)] * SCALE
                    plsc.store_scatter(
                        wrows, [bvecs[b], kvecs[j], rvecs[j], cc], v)

            pltpu.async_copy(
                wrows.at[b, :, :, pl.ds(0, CHUNK)],
                out_hbm.at[pl.ds(d1 * 8, 8), tc, :, :],
                wsem.at[b])

            @pl.when(ql + NBUF < n_chunks)
            def _():
                pltpu.async_copy(
                    lut_hbm.at[idx_v.at[ql + NBUF]], grows.at[b], gsem.at[b])
        return carry

    lax.fori_loop(0, n_chunks // NBUF, outer, 0)

    for b in range(NBUF):
        pltpu.make_async_copy(
            wrows.at[b, :, :, pl.ds(0, CHUNK)],
            out_hbm.at[pl.ds(0, 8), 0, :, :], wsem.at[b]).wait()


def _lut_transpose_body(xt_ref, o_ref):
    t = xt_ref[...].T
    t3 = t.reshape(t.shape[0] // 2, 2, t.shape[1])
    o_ref[...] = jnp.concatenate([t3[:, 0, :], t3[:, 1, :]], axis=-1)


def _linearize_lut(lut):
    v = lut.shape[0]
    vb = 3328
    lut2 = pl.pallas_call(
        _lut_transpose_body,
        grid=(pl.cdiv(v, vb),),
        in_specs=[pl.BlockSpec((D_MODEL, vb), lambda g: (0, g))],
        out_specs=pl.BlockSpec((vb // 2, 2 * D_MODEL), lambda g: (g, 0)),
        out_shape=jax.ShapeDtypeStruct((v // 2, 2 * D_MODEL), jnp.float32),
    )(lut.T)
    return lut2.reshape(v, D_MODEL)


def kernel(x, lut):
    S, T = x.shape
    xq = x.T.reshape(T * S // CHUNK, CHUNK)
    k = pl.kernel(
        _embed_body,
        out_type=jax.ShapeDtypeStruct((T * 8, S // CHUNK, 8, CHUNK),
                                      jnp.float32),
        mesh=plsc.VectorSubcoreMesh(core_axis_name="c", subcore_axis_name="s"),
        scratch_types=[
            pltpu.VMEM((T * S // (CHUNK * NW), CHUNK), jnp.int32),
            pltpu.VMEM((NBUF, CHUNK, D_MODEL), jnp.float32),
            pltpu.VMEM((NBUF, 8, 8, CHUNK + 1), jnp.float32),
            pltpu.SemaphoreType.DMA((NBUF,)),
            pltpu.SemaphoreType.DMA((NBUF,)),
        ],
        compiler_params=pltpu.CompilerParams(use_tc_tiling_on_sc=False,
                                             needs_layout_passes=False),
    )
    out4 = k(xq, _linearize_lut(lut))
    return (out4.reshape(T, 8, S // CHUNK, 8, CHUNK)
            .transpose(2, 4, 0, 1, 3)
            .reshape(S, T, D_MODEL))


# speedup vs baseline: 4.5903x; 1.4215x over previous
import math

import jax
import jax.numpy as jnp
from jax import lax
from jax.experimental import pallas as pl
from jax.experimental.pallas import tpu as pltpu
from jax.experimental.pallas import tpu_sc as plsc

D_MODEL = 64
SCALE = math.sqrt(D_MODEL)
NC, NS = 2, 16
NW = NC * NS
LANES = 16
CHUNK = 128
NBUF = 4


def _embed_body(x_hbm, lut_hbm, out_hbm, idx_v, grows, wrows, gsem, wsem):
    wid = lax.axis_index("s") * NC + lax.axis_index("c")
    n_chunks = x_hbm.shape[0] // NW
    q0 = wid * n_chunks

    pltpu.sync_copy(x_hbm.at[pl.ds(q0, n_chunks), :], idx_v)

    @plsc.parallel_loop(0, n_chunks, step=1, unroll=2)
    def _(r):
        for kk in range(CHUNK // LANES):
            sl = pl.ds(kk * LANES, LANES)
            v = idx_v[r, sl]
            rem = lax.bitwise_and(v, 4095)
            h = lax.shift_right_logical(rem, 11)
            jl = lax.bitwise_and(rem, 2047)
            idx_v[r, sl] = (v - rem) + 2 * jl + h

    for b in range(NBUF):
        pltpu.async_copy(lut_hbm.at[idx_v.at[b]], grows.at[b], gsem.at[b])

    iota = lax.iota(jnp.int32, LANES)
    kvecs = [lax.shift_right_logical(iota + 16 * j, 3) for j in range(4)]
    rvecs = [lax.bitwise_and(iota + 16 * j, 7) for j in range(4)]
    bvecs = [jnp.full((LANES,), b, jnp.int32) for b in range(NBUF)]

    def outer(t, carry):
        for b in range(NBUF):
            ql = NBUF * t + b
            q = q0 + ql
            d1 = lax.shift_right_logical(q, 5)
            tc = lax.bitwise_and(q, 31)

            pltpu.make_async_copy(
                lut_hbm.at[idx_v.at[b]], grows.at[b], gsem.at[b]).wait()

            @pl.when(ql >= NBUF)
            def _():
                pltpu.make_async_copy(
                    wrows.at[b, :, :, pl.ds(0, CHUNK)],
                    out_hbm.at[pl.ds(0, 8), 0, :, :],
                    wsem.at[b]).wait()

            @plsc.parallel_loop(0, CHUNK, step=1, unroll=2)
            def _(i):
                cc = jnp.full((LANES,), i, jnp.int32)
                for j in range(4):
                    v = grows[b, i, pl.ds(16 * j, LANES)] * SCALE
                    plsc.store_scatter(
                        wrows, [bvecs[b], kvecs[j], rvecs[j], cc], v)

            pltpu.async_copy(
                wrows.at[b, :, :, pl.ds(0, CHUNK)],
                out_hbm.at[pl.ds(d1 * 8, 8), tc, :, :],
                wsem.at[b])

            @pl.when(ql + NBUF < n_chunks)
            def _():
                pltpu.async_copy(
                    lut_hbm.at[idx_v.at[ql + NBUF]], grows.at[b], gsem.at[b])
        return carry

    lax.fori_loop(0, n_chunks // NBUF, outer, 0)

    for b in range(NBUF):
        pltpu.make_async_copy(
            wrows.at[b, :, :, pl.ds(0, CHUNK)],
            out_hbm.at[pl.ds(0, 8), 0, :, :], wsem.at[b]).wait()


HB = 2048


def _lut_transpose_body(a_ref, b_ref, o_ref):
    o_ref[...] = jnp.concatenate([a_ref[...], b_ref[...]], axis=0).T


def _linearize_lut(lut):
    v = lut.shape[0]
    grid = pl.cdiv(v, 2 * HB)
    nhalf = 2 * grid - 1
    lut2 = pl.pallas_call(
        _lut_transpose_body,
        grid=(grid,),
        in_specs=[
            pl.BlockSpec((D_MODEL, HB), lambda g: (0, 2 * g)),
            pl.BlockSpec((D_MODEL, HB),
                         lambda g: (0, jnp.minimum(2 * g + 1, nhalf - 1))),
        ],
        out_specs=pl.BlockSpec((HB, 2 * D_MODEL), lambda g: (g, 0)),
        out_shape=jax.ShapeDtypeStruct((grid * HB, 2 * D_MODEL), jnp.float32),
    )(lut.T, lut.T)
    return lut2.reshape(2 * grid * HB, D_MODEL)


def kernel(x, lut):
    S, T = x.shape
    xq = x.T.reshape(T * S // CHUNK, CHUNK)
    k = pl.kernel(
        _embed_body,
        out_type=jax.ShapeDtypeStruct((T * 8, S // CHUNK, 8, CHUNK),
                                      jnp.float32),
        mesh=plsc.VectorSubcoreMesh(core_axis_name="c", subcore_axis_name="s"),
        scratch_types=[
            pltpu.VMEM((T * S // (CHUNK * NW), CHUNK), jnp.int32),
            pltpu.VMEM((NBUF, CHUNK, D_MODEL), jnp.float32),
            pltpu.VMEM((NBUF, 8, 8, CHUNK + 1), jnp.float32),
            pltpu.SemaphoreType.DMA((NBUF,)),
            pltpu.SemaphoreType.DMA((NBUF,)),
        ],
        compiler_params=pltpu.CompilerParams(use_tc_tiling_on_sc=False,
                                             needs_layout_passes=False),
    )
    out4 = k(xq, _linearize_lut(lut))
    return (out4.reshape(T, 8, S // CHUNK, 8, CHUNK)
            .transpose(2, 4, 0, 1, 3)
            .reshape(S, T, D_MODEL))


# speedup vs baseline: 5.4954x; 1.1972x over previous
import math

import jax
import jax.numpy as jnp
from jax import lax
from jax.experimental import pallas as pl
from jax.experimental.pallas import tpu as pltpu
from jax.experimental.pallas import tpu_sc as plsc

D_MODEL = 64
SCALE = math.sqrt(D_MODEL)
NC, NS = 2, 16
NW = NC * NS
LANES = 16
CHUNK = 128
NBUF = 4


def _embed_body(x_hbm, lut_hbm, out_hbm, idx_v, grows, wrows, gsem, wsem):
    wid = lax.axis_index("s") * NC + lax.axis_index("c")
    n_chunks = x_hbm.shape[0] // NW
    q0 = wid * n_chunks

    pltpu.sync_copy(x_hbm.at[pl.ds(q0, n_chunks), :], idx_v)

    @plsc.parallel_loop(0, n_chunks, step=1, unroll=2)
    def _(r):
        for kk in range(CHUNK // LANES):
            sl = pl.ds(kk * LANES, LANES)
            v = idx_v[r, sl]
            rem = lax.bitwise_and(v, 2 * HB - 1)
            h = lax.shift_right_logical(rem, 12)
            jl = lax.bitwise_and(rem, HB - 1)
            idx_v[r, sl] = (v - rem) + 2 * jl + h

    for b in range(NBUF):
        pltpu.async_copy(lut_hbm.at[idx_v.at[b]], grows.at[b], gsem.at[b])

    iota = lax.iota(jnp.int32, LANES)
    kvecs = [lax.shift_right_logical(iota + 16 * j, 3) for j in range(4)]
    rvecs = [lax.bitwise_and(iota + 16 * j, 7) for j in range(4)]
    bvecs = [jnp.full((LANES,), b, jnp.int32) for b in range(NBUF)]

    def outer(t, carry):
        for b in range(NBUF):
            ql = NBUF * t + b
            q = q0 + ql
            d1 = lax.shift_right_logical(q, 5)
            tc = lax.bitwise_and(q, 31)

            pltpu.make_async_copy(
                lut_hbm.at[idx_v.at[b]], grows.at[b], gsem.at[b]).wait()

            @pl.when(ql >= NBUF)
            def _():
                pltpu.make_async_copy(
                    wrows.at[b, :, :, pl.ds(0, CHUNK)],
                    out_hbm.at[pl.ds(0, 8), 0, :, :],
                    wsem.at[b]).wait()

            @plsc.parallel_loop(0, CHUNK, step=1, unroll=2)
            def _(i):
                cc = jnp.full((LANES,), i, jnp.int32)
                for j in range(4):
                    v = grows[b, i, pl.ds(16 * j, LANES)] * SCALE
                    plsc.store_scatter(
                        wrows, [bvecs[b], kvecs[j], rvecs[j], cc], v)

            pltpu.async_copy(
                wrows.at[b, :, :, pl.ds(0, CHUNK)],
                out_hbm.at[pl.ds(d1 * 8, 8), tc, :, :],
                wsem.at[b])

            @pl.when(ql + NBUF < n_chunks)
            def _():
                pltpu.async_copy(
                    lut_hbm.at[idx_v.at[ql + NBUF]], grows.at[b], gsem.at[b])
        return carry

    lax.fori_loop(0, n_chunks // NBUF, outer, 0)

    for b in range(NBUF):
        pltpu.make_async_copy(
            wrows.at[b, :, :, pl.ds(0, CHUNK)],
            out_hbm.at[pl.ds(0, 8), 0, :, :], wsem.at[b]).wait()


HB = 4096


def _lut_transpose_body(a_ref, b_ref, o_ref):
    o_ref[...] = jnp.concatenate([a_ref[...], b_ref[...]], axis=0).T


def _linearize_lut(lut):
    v = lut.shape[0]
    grid = pl.cdiv(v, 2 * HB)
    nhalf = 2 * grid - 1
    lut2 = pl.pallas_call(
        _lut_transpose_body,
        grid=(grid,),
        in_specs=[
            pl.BlockSpec((D_MODEL, HB), lambda g: (0, 2 * g)),
            pl.BlockSpec((D_MODEL, HB),
                         lambda g: (0, jnp.minimum(2 * g + 1, nhalf - 1))),
        ],
        out_specs=pl.BlockSpec((HB, 2 * D_MODEL), lambda g: (g, 0)),
        out_shape=jax.ShapeDtypeStruct((grid * HB, 2 * D_MODEL), jnp.float32),
    )(lut.T, lut.T)
    return lut2.reshape(2 * grid * HB, D_MODEL)


def kernel(x, lut):
    S, T = x.shape
    xq = x.T.reshape(T * S // CHUNK, CHUNK)
    k = pl.kernel(
        _embed_body,
        out_type=jax.ShapeDtypeStruct((T * 8, S // CHUNK, 8, CHUNK),
                                      jnp.float32),
        mesh=plsc.VectorSubcoreMesh(core_axis_name="c", subcore_axis_name="s"),
        scratch_types=[
            pltpu.VMEM((T * S // (CHUNK * NW), CHUNK), jnp.int32),
            pltpu.VMEM((NBUF, CHUNK, D_MODEL), jnp.float32),
            pltpu.VMEM((NBUF, 8, 8, CHUNK + 1), jnp.float32),
            pltpu.SemaphoreType.DMA((NBUF,)),
            pltpu.SemaphoreType.DMA((NBUF,)),
        ],
        compiler_params=pltpu.CompilerParams(use_tc_tiling_on_sc=False,
                                             needs_layout_passes=False),
    )
    out4 = k(xq, _linearize_lut(lut))
    return (out4.reshape(T, 8, S // CHUNK, 8, CHUNK)
            .transpose(2, 4, 0, 1, 3)
            .reshape(S, T, D_MODEL))


# speedup vs baseline: 5.9580x; 1.0842x over previous
import math

import jax
import jax.numpy as jnp
from jax import lax
from jax.experimental import pallas as pl
from jax.experimental.pallas import tpu as pltpu
from jax.experimental.pallas import tpu_sc as plsc

D_MODEL = 64
SCALE = math.sqrt(D_MODEL)
NC, NS = 2, 16
NW = NC * NS
LANES = 16
CHUNK = 128
NBUF = 4


def _embed_body(x_hbm, lut_hbm, out_hbm, idx_v, grows, wrows, gsem, wsem):
    wid = lax.axis_index("s") * NC + lax.axis_index("c")
    n_chunks = x_hbm.shape[0] // NW
    q0 = wid * n_chunks

    pltpu.sync_copy(x_hbm.at[pl.ds(q0, n_chunks), :], idx_v)

    @plsc.parallel_loop(0, n_chunks, step=1, unroll=2)
    def _(r):
        for kk in range(CHUNK // LANES):
            sl = pl.ds(kk * LANES, LANES)
            v = idx_v[r, sl]
            rem = lax.bitwise_and(v, 2 * HB - 1)
            h = lax.shift_right_logical(rem, 13)
            jl = lax.bitwise_and(rem, HB - 1)
            idx_v[r, sl] = (v - rem) + 2 * jl + h

    for b in range(NBUF):
        pltpu.async_copy(lut_hbm.at[idx_v.at[b]], grows.at[b], gsem.at[b])

    iota = lax.iota(jnp.int32, LANES)
    kvecs = [lax.shift_right_logical(iota + 16 * j, 3) for j in range(4)]
    rvecs = [lax.bitwise_and(iota + 16 * j, 7) for j in range(4)]
    bvecs = [jnp.full((LANES,), b, jnp.int32) for b in range(NBUF)]

    def outer(t, carry):
        for b in range(NBUF):
            ql = NBUF * t + b
            q = q0 + ql
            d1 = lax.shift_right_logical(q, 5)
            tc = lax.bitwise_and(q, 31)

            pltpu.make_async_copy(
                lut_hbm.at[idx_v.at[b]], grows.at[b], gsem.at[b]).wait()

            @pl.when(ql >= NBUF)
            def _():
                pltpu.make_async_copy(
                    wrows.at[b, :, :, pl.ds(0, CHUNK)],
                    out_hbm.at[pl.ds(0, 8), 0, :, :],
                    wsem.at[b]).wait()

            @plsc.parallel_loop(0, CHUNK, step=1, unroll=2)
            def _(i):
                cc = jnp.full((LANES,), i, jnp.int32)
                for j in range(4):
                    v = grows[b, i, pl.ds(16 * j, LANES)] * SCALE
                    plsc.store_scatter(
                        wrows, [bvecs[b], kvecs[j], rvecs[j], cc], v)

            pltpu.async_copy(
                wrows.at[b, :, :, pl.ds(0, CHUNK)],
                out_hbm.at[pl.ds(d1 * 8, 8), tc, :, :],
                wsem.at[b])

            @pl.when(ql + NBUF < n_chunks)
            def _():
                pltpu.async_copy(
                    lut_hbm.at[idx_v.at[ql + NBUF]], grows.at[b], gsem.at[b])
        return carry

    lax.fori_loop(0, n_chunks // NBUF, outer, 0)

    for b in range(NBUF):
        pltpu.make_async_copy(
            wrows.at[b, :, :, pl.ds(0, CHUNK)],
            out_hbm.at[pl.ds(0, 8), 0, :, :], wsem.at[b]).wait()


HB = 8192


def _lut_transpose_body(a_ref, b_ref, o_ref):
    o_ref[...] = jnp.concatenate([a_ref[...], b_ref[...]], axis=0).T


def _linearize_lut(lut):
    v = lut.shape[0]
    grid = pl.cdiv(v, 2 * HB)
    nhalf = 2 * grid - 1
    lut2 = pl.pallas_call(
        _lut_transpose_body,
        grid=(grid,),
        in_specs=[
            pl.BlockSpec((D_MODEL, HB), lambda g: (0, 2 * g)),
            pl.BlockSpec((D_MODEL, HB),
                         lambda g: (0, jnp.minimum(2 * g + 1, nhalf - 1))),
        ],
        out_specs=pl.BlockSpec((HB, 2 * D_MODEL), lambda g: (g, 0)),
        out_shape=jax.ShapeDtypeStruct((grid * HB, 2 * D_MODEL), jnp.float32),
    )(lut.T, lut.T)
    return lut2.reshape(2 * grid * HB, D_MODEL)


def kernel(x, lut):
    S, T = x.shape
    xq = x.T.reshape(T * S // CHUNK, CHUNK)
    k = pl.kernel(
        _embed_body,
        out_type=jax.ShapeDtypeStruct((T * 8, S // CHUNK, 8, CHUNK),
                                      jnp.float32),
        mesh=plsc.VectorSubcoreMesh(core_axis_name="c", subcore_axis_name="s"),
        scratch_types=[
            pltpu.VMEM((T * S // (CHUNK * NW), CHUNK), jnp.int32),
            pltpu.VMEM((NBUF, CHUNK, D_MODEL), jnp.float32),
            pltpu.VMEM((NBUF, 8, 8, CHUNK + 1), jnp.float32),
            pltpu.SemaphoreType.DMA((NBUF,)),
            pltpu.SemaphoreType.DMA((NBUF,)),
        ],
        compiler_params=pltpu.CompilerParams(use_tc_tiling_on_sc=False,
                                             needs_layout_passes=False),
    )
    out4 = k(xq, _linearize_lut(lut))
    return (out4.reshape(T, 8, S // CHUNK, 8, CHUNK)
            .transpose(2, 4, 0, 1, 3)
            .reshape(S, T, D_MODEL))


# speedup vs baseline: 6.0479x; 1.0151x over previous
import math

import jax
import jax.numpy as jnp
from jax import lax
from jax.experimental import pallas as pl
from jax.experimental.pallas import tpu as pltpu
from jax.experimental.pallas import tpu_sc as plsc

D_MODEL = 64
SCALE = math.sqrt(D_MODEL)
NC, NS = 2, 16
NW = NC * NS
LANES = 16
CHUNK = 128
NBUF = 4


def _embed_body(x_hbm, lut_hbm, out_hbm, idx_v, grows, wrows, gsem, wsem):
    wid = lax.axis_index("s") * NC + lax.axis_index("c")
    n_chunks = x_hbm.shape[0] // NW
    q0 = wid * n_chunks

    pltpu.sync_copy(x_hbm.at[pl.ds(q0, n_chunks), :], idx_v)

    @plsc.parallel_loop(0, n_chunks, step=1, unroll=2)
    def _(r):
        for kk in range(CHUNK // LANES):
            sl = pl.ds(kk * LANES, LANES)
            v = idx_v[r, sl]
            rem = lax.bitwise_and(v, 2 * HB - 1)
            h = lax.shift_right_logical(rem, 14)
            jl = lax.bitwise_and(rem, HB - 1)
            idx_v[r, sl] = (v - rem) + 2 * jl + h

    for b in range(NBUF):
        pltpu.async_copy(lut_hbm.at[idx_v.at[b]], grows.at[b], gsem.at[b])

    iota = lax.iota(jnp.int32, LANES)
    kvecs = [lax.shift_right_logical(iota + 16 * j, 3) for j in range(4)]
    rvecs = [lax.bitwise_and(iota + 16 * j, 7) for j in range(4)]
    bvecs = [jnp.full((LANES,), b, jnp.int32) for b in range(NBUF)]

    def outer(t, carry):
        for b in range(NBUF):
            ql = NBUF * t + b
            q = q0 + ql
            d1 = lax.shift_right_logical(q, 5)
            tc = lax.bitwise_and(q, 31)

            pltpu.make_async_copy(
                lut_hbm.at[idx_v.at[b]], grows.at[b], gsem.at[b]).wait()

            @pl.when(ql >= NBUF)
            def _():
                pltpu.make_async_copy(
                    wrows.at[b, :, :, pl.ds(0, CHUNK)],
                    out_hbm.at[pl.ds(0, 8), 0, :, :],
                    wsem.at[b]).wait()

            @plsc.parallel_loop(0, CHUNK, step=1, unroll=2)
            def _(i):
                cc = jnp.full((LANES,), i, jnp.int32)
                for j in range(4):
                    v = grows[b, i, pl.ds(16 * j, LANES)] * SCALE
                    plsc.store_scatter(
                        wrows, [bvecs[b], kvecs[j], rvecs[j], cc], v)

            pltpu.async_copy(
                wrows.at[b, :, :, pl.ds(0, CHUNK)],
                out_hbm.at[pl.ds(d1 * 8, 8), tc, :, :],
                wsem.at[b])

            @pl.when(ql + NBUF < n_chunks)
            def _():
                pltpu.async_copy(
                    lut_hbm.at[idx_v.at[ql + NBUF]], grows.at[b], gsem.at[b])
        return carry

    lax.fori_loop(0, n_chunks // NBUF, outer, 0)

    for b in range(NBUF):
        pltpu.make_async_copy(
            wrows.at[b, :, :, pl.ds(0, CHUNK)],
            out_hbm.at[pl.ds(0, 8), 0, :, :], wsem.at[b]).wait()


HB = 16384


def _lut_transpose_body(a_ref, b_ref, o_ref):
    o_ref[...] = jnp.concatenate([a_ref[...], b_ref[...]], axis=0).T


def _linearize_lut(lut):
    v = lut.shape[0]
    grid = pl.cdiv(v, 2 * HB)
    nhalf = pl.cdiv(v, HB)
    lut2 = pl.pallas_call(
        _lut_transpose_body,
        grid=(grid,),
        in_specs=[
            pl.BlockSpec((D_MODEL, HB), lambda g: (0, 2 * g)),
            pl.BlockSpec((D_MODEL, HB),
                         lambda g: (0, jnp.minimum(2 * g + 1, nhalf - 1))),
        ],
        out_specs=pl.BlockSpec((HB, 2 * D_MODEL), lambda g: (g, 0)),
        out_shape=jax.ShapeDtypeStruct((grid * HB, 2 * D_MODEL), jnp.float32),
    )(lut.T, lut.T)
    return lut2.reshape(2 * grid * HB, D_MODEL)


def kernel(x, lut):
    S, T = x.shape
    xq = x.T.reshape(T * S // CHUNK, CHUNK)
    k = pl.kernel(
        _embed_body,
        out_type=jax.ShapeDtypeStruct((T * 8, S // CHUNK, 8, CHUNK),
                                      jnp.float32),
        mesh=plsc.VectorSubcoreMesh(core_axis_name="c", subcore_axis_name="s"),
        scratch_types=[
            pltpu.VMEM((T * S // (CHUNK * NW), CHUNK), jnp.int32),
            pltpu.VMEM((NBUF, CHUNK, D_MODEL), jnp.float32),
            pltpu.VMEM((NBUF, 8, 8, CHUNK + 1), jnp.float32),
            pltpu.SemaphoreType.DMA((NBUF,)),
            pltpu.SemaphoreType.DMA((NBUF,)),
        ],
        compiler_params=pltpu.CompilerParams(use_tc_tiling_on_sc=False,
                                             needs_layout_passes=False),
    )
    out4 = k(xq, _linearize_lut(lut))
    return (out4.reshape(T, 8, S // CHUNK, 8, CHUNK)
            .transpose(2, 4, 0, 1, 3)
            .reshape(S, T, D_MODEL))
